# baseline (speedup 1.0000x reference)
import jax
import jax.numpy as jnp
from jax.experimental import pallas as pl
from jax.experimental.pallas import tpu as pltpu

_HALF = 6.0
_MAXC = 63.0
_M_EMA = 0.999
_B, _C, _H, _W, _L = 64, 256, 64, 64, 21
_CORES = 2
_BPC = _B // _CORES


def _pool_one(f_blk, pre_t_ref, onehot):
    x = jnp.sum(pre_t_ref[:, 0, :] * onehot, axis=1, keepdims=True)
    y = jnp.sum(pre_t_ref[:, 1, :] * onehot, axis=1, keepdims=True)
    down = jnp.maximum(y - _HALF, 0.0).astype(jnp.int32)
    left = jnp.maximum(x - _HALF, 0.0).astype(jnp.int32)
    upper = jnp.minimum(y + _HALF, _MAXC).astype(jnp.int32)
    right = jnp.minimum(x + _HALF, _MAXC).astype(jnp.int32)
    j = jax.lax.broadcasted_iota(jnp.int32, (_L, _H * _W), 1)
    h = j >> 6
    w = j & (_W - 1)
    mask = ((h >= left) & (h < right) & (w >= down) & (w < upper)
            ).astype(jnp.float32)
    s = ((upper - down + 1) * (right - left + 1)).astype(jnp.float32)
    f = f_blk[0]
    f_hi = jax.lax.bitcast_convert_type(
        jax.lax.bitcast_convert_type(f, jnp.uint32) & jnp.uint32(0xFFFF0000),
        jnp.float32)
    f_lo = f - f_hi
    pooled = (jnp.dot(mask, f_hi, preferred_element_type=jnp.float32)
              + jnp.dot(mask, f_lo, preferred_element_type=jnp.float32))
    return pooled / s


def _pool_kernel(f1_ref, f2_ref, pre1_ref, pre2_ref, o1_ref, o2_ref):
    k = pl.program_id(0)
    i = pl.program_id(1)

    @pl.when(i == 0)
    def _():
        o1_ref[...] = jnp.zeros_like(o1_ref)
        o2_ref[...] = jnp.zeros_like(o2_ref)

    b = k * _BPC + i
    bi = jax.lax.broadcasted_iota(jnp.int32, (1, _B), 1)
    onehot = (bi == b).astype(jnp.float32)
    o1_ref[...] += _pool_one(f1_ref, pre1_ref, onehot)[None]
    o2_ref[...] += _pool_one(f2_ref, pre2_ref, onehot)[None]


def _finalize_kernel(p1_ref, p2_ref, fea1_ref, fea2_ref, o_ref):
    m1 = (p1_ref[0] + p1_ref[1]) * (1.0 / _B)
    m2 = (p2_ref[0] + p2_ref[1]) * (1.0 / _B)
    fea_c1 = _M_EMA * m1 + (1.0 - _M_EMA) * fea1_ref[...]
    fea_c2 = _M_EMA * m2 + (1.0 - _M_EMA) * fea2_ref[...]
    z = fea_c1 - jnp.max(fea_c1, axis=1, keepdims=True)
    log_p = z - jnp.log(jnp.sum(jnp.exp(z), axis=1, keepdims=True))
    q = fea_c2 / jnp.sum(fea_c2, axis=1, keepdims=True)
    kl = jnp.where(q > 0, q * (jnp.log(jnp.where(q > 0, q, 1.0)) - log_p), 0.0)
    o_ref[0, 0] = jnp.sum(kl) * (1.0 / _L)


def kernel(f1, f2, pre1, pre2, fea1, fea2):
    f1v = f1.transpose(0, 2, 3, 1).reshape(_B, _H * _W, _C)
    f2v = f2.transpose(0, 2, 3, 1).reshape(_B, _H * _W, _C)
    pre1t = pre1.transpose(1, 2, 0)
    pre2t = pre2.transpose(1, 2, 0)

    f_spec = pl.BlockSpec((1, _H * _W, _C), lambda k, i: (k * _BPC + i, 0, 0))
    pre_spec = pl.BlockSpec((_L, 2, _B), lambda k, i: (0, 0, 0))
    out_spec = pl.BlockSpec((1, _L, _C), lambda k, i: (k, 0, 0))

    p1, p2 = pl.pallas_call(
        _pool_kernel,
        grid=(_CORES, _BPC),
        in_specs=[f_spec, f_spec, pre_spec, pre_spec],
        out_specs=[out_spec, out_spec],
        out_shape=[jax.ShapeDtypeStruct((_CORES, _L, _C), jnp.float32)] * 2,
        compiler_params=pltpu.CompilerParams(
            dimension_semantics=("parallel", "arbitrary")),
        name="window_pool",
    )(f1v, f2v, pre1t, pre2t)

    out = pl.pallas_call(
        _finalize_kernel,
        in_specs=[pl.BlockSpec(memory_space=pltpu.VMEM)] * 4,
        out_specs=pl.BlockSpec(memory_space=pltpu.SMEM),
        out_shape=jax.ShapeDtypeStruct((1, 1), jnp.float32),
        name="ema_kl_finalize",
    )(p1, p2, fea1, fea2)
    return out[0, 0]


# speedup vs baseline: 1.0930x; 1.0930x over previous
import jax
import jax.numpy as jnp
from jax.experimental import pallas as pl
from jax.experimental.pallas import tpu as pltpu

_HALF = 6.0
_MAXC = 63.0
_M_EMA = 0.999
_B, _C, _H, _W, _L = 64, 256, 64, 64, 21
_CORES = 2
_BPC = _B // _CORES


def _pool_one(f_blk, pre_t_ref, onehot):
    x = jnp.sum(pre_t_ref[:, 0, :] * onehot, axis=1, keepdims=True)
    y = jnp.sum(pre_t_ref[:, 1, :] * onehot, axis=1, keepdims=True)
    down = jnp.maximum(y - _HALF, 0.0).astype(jnp.int32)
    left = jnp.maximum(x - _HALF, 0.0).astype(jnp.int32)
    upper = jnp.minimum(y + _HALF, _MAXC).astype(jnp.int32)
    right = jnp.minimum(x + _HALF, _MAXC).astype(jnp.int32)
    j = jax.lax.broadcasted_iota(jnp.int32, (_L, _H * _W), 1)
    h = j >> 6
    w = j & (_W - 1)
    mask = ((h >= left) & (h < right) & (w >= down) & (w < upper)
            ).astype(jnp.float32)
    s = ((upper - down + 1) * (right - left + 1)).astype(jnp.float32)
    pooled = jnp.dot(mask, f_blk[0], preferred_element_type=jnp.float32)
    return pooled / s


def _pool_kernel(f1_ref, f2_ref, pre1_ref, pre2_ref, o1_ref, o2_ref):
    k = pl.program_id(0)
    i = pl.program_id(1)

    @pl.when(i == 0)
    def _():
        o1_ref[...] = jnp.zeros_like(o1_ref)
        o2_ref[...] = jnp.zeros_like(o2_ref)

    b = k * _BPC + i
    bi = jax.lax.broadcasted_iota(jnp.int32, (1, _B), 1)
    onehot = (bi == b).astype(jnp.float32)
    o1_ref[...] += _pool_one(f1_ref, pre1_ref, onehot)[None]
    o2_ref[...] += _pool_one(f2_ref, pre2_ref, onehot)[None]


def _finalize_kernel(p1_ref, p2_ref, fea1_ref, fea2_ref, o_ref):
    m1 = (p1_ref[0] + p1_ref[1]) * (1.0 / _B)
    m2 = (p2_ref[0] + p2_ref[1]) * (1.0 / _B)
    fea_c1 = _M_EMA * m1 + (1.0 - _M_EMA) * fea1_ref[...]
    fea_c2 = _M_EMA * m2 + (1.0 - _M_EMA) * fea2_ref[...]
    z = fea_c1 - jnp.max(fea_c1, axis=1, keepdims=True)
    log_p = z - jnp.log(jnp.sum(jnp.exp(z), axis=1, keepdims=True))
    q = fea_c2 / jnp.sum(fea_c2, axis=1, keepdims=True)
    kl = jnp.where(q > 0, q * (jnp.log(jnp.where(q > 0, q, 1.0)) - log_p), 0.0)
    o_ref[0, 0] = jnp.sum(kl) * (1.0 / _L)


def kernel(f1, f2, pre1, pre2, fea1, fea2):
    f1v = f1.transpose(0, 2, 3, 1).reshape(_B, _H * _W, _C)
    f2v = f2.transpose(0, 2, 3, 1).reshape(_B, _H * _W, _C)
    pre1t = pre1.transpose(1, 2, 0)
    pre2t = pre2.transpose(1, 2, 0)

    f_spec = pl.BlockSpec((1, _H * _W, _C), lambda k, i: (k * _BPC + i, 0, 0))
    pre_spec = pl.BlockSpec((_L, 2, _B), lambda k, i: (0, 0, 0))
    out_spec = pl.BlockSpec((1, _L, _C), lambda k, i: (k, 0, 0))

    p1, p2 = pl.pallas_call(
        _pool_kernel,
        grid=(_CORES, _BPC),
        in_specs=[f_spec, f_spec, pre_spec, pre_spec],
        out_specs=[out_spec, out_spec],
        out_shape=[jax.ShapeDtypeStruct((_CORES, _L, _C), jnp.float32)] * 2,
        compiler_params=pltpu.CompilerParams(
            dimension_semantics=("parallel", "arbitrary")),
        name="window_pool",
    )(f1v, f2v, pre1t, pre2t)

    out = pl.pallas_call(
        _finalize_kernel,
        in_specs=[pl.BlockSpec(memory_space=pltpu.VMEM)] * 4,
        out_specs=pl.BlockSpec(memory_space=pltpu.SMEM),
        out_shape=jax.ShapeDtypeStruct((1, 1), jnp.float32),
        name="ema_kl_finalize",
    )(p1, p2, fea1, fea2)
    return out[0, 0]
